# baseline (device time: 389815 ns/iter reference)
import numpy as np
import jax
import jax.numpy as jnp
from jax import lax
from jax.experimental import pallas as pl
from jax.experimental.pallas import tpu as pltpu

N_DEV = 4
S = 2048
D = 1024
H = 8
DH = 128
PAIR = 2 * DH
SCALE = 0.08838834764831843
QBLK = 512


def _rope_tables():
    inv = 1.0 / (10000.0 ** (np.arange(0, DH, 2) / DH))
    pos = np.arange(S)[:, None] * inv[None, :]
    cos = np.repeat(np.cos(pos), 2, axis=-1).astype(np.float32)
    sin = np.repeat(np.sin(pos), 2, axis=-1)
    sign = np.tile(np.array([-1.0, 1.0]), DH // 2)
    sin_alt = (sin * sign[None, :]).astype(np.float32)
    return np.tile(cos, (1, 2)), np.tile(sin_alt, (1, 2))


def _swap_matrix():
    p = np.zeros((PAIR, PAIR), np.float32)
    idx = np.arange(PAIR)
    p[idx, idx ^ 1] = 1.0
    return p


def kernel(x, Wq, Wk, Wv, Wo):
    x2 = x[0].astype(jnp.bfloat16)
    w_own = jnp.stack([Wq, Wk, Wv, Wo]).astype(jnp.bfloat16)
    cos_np, sin_np = _rope_tables()
    cos_t = jnp.asarray(cos_np)
    sin_t = jnp.asarray(sin_np)
    p_swap = jnp.asarray(_swap_matrix(), dtype=jnp.bfloat16)

    def body(x_ref, w_ref, cos_ref, sin_ref, p_ref, out_ref,
             comm_ref, send_sems, recv_sems, credit_sem):
        my = lax.axis_index("i")
        left = (my + N_DEV - 1) % N_DEV
        right = (my + 1) % N_DEV

        barrier = pltpu.get_barrier_semaphore()
        for nbr in (left, right):
            pl.semaphore_signal(barrier, inc=1, device_id=(nbr,),
                                device_id_type=pl.DeviceIdType.MESH)
        pl.semaphore_wait(barrier, 2)

        for qb in range(S // QBLK):
            out_ref[0, pl.ds(qb * QBLK, QBLK), :] = (
                jnp.zeros((QBLK, D), jnp.float32))
        xv = x_ref[...]
        pv = p_ref[...]

        for h in range(N_DEV):
            if h < N_DEV - 1:
                if h == 2:
                    pl.semaphore_wait(credit_sem, 1)
                rdma = pltpu.make_async_remote_copy(
                    src_ref=(w_ref if h == 0 else comm_ref.at[(h - 1) % 2]),
                    dst_ref=comm_ref.at[h % 2],
                    send_sem=send_sems.at[h],
                    recv_sem=recv_sems.at[h],
                    device_id=(right,),
                    device_id_type=pl.DeviceIdType.MESH,
                )
                rdma.start()

            def pair_body(hp, carry, _h=h):
                off = hp * PAIR
                slot = (_h - 1) % 2
                if _h == 0:
                    wq_p = w_ref[0, :, pl.ds(off, PAIR)]
                    wk_p = w_ref[1, :, pl.ds(off, PAIR)]
                    wv_p = w_ref[2, :, pl.ds(off, PAIR)]
                    wo_p = w_ref[3, pl.ds(off, PAIR), :]
                else:
                    wq_p = comm_ref[slot, 0, :, pl.ds(off, PAIR)]
                    wk_p = comm_ref[slot, 1, :, pl.ds(off, PAIR)]
                    wv_p = comm_ref[slot, 2, :, pl.ds(off, PAIR)]
                    wo_p = comm_ref[slot, 3, pl.ds(off, PAIR), :]

                cos_f = cos_ref[...]
                sin_f = sin_ref[...]

                k_raw = lax.dot_general(xv, wk_p, (((1,), (0,)), ((), ())),
                                        preferred_element_type=jnp.float32)
                k_sw = lax.dot_general(k_raw.astype(jnp.bfloat16), pv,
                                       (((1,), (0,)), ((), ())),
                                       preferred_element_type=jnp.float32)
                k_p = (k_raw * cos_f + k_sw * sin_f).astype(jnp.bfloat16)
                v_p = lax.dot_general(
                    xv, wv_p, (((1,), (0,)), ((), ())),
                    preferred_element_type=jnp.float32).astype(jnp.bfloat16)

                for qb in range(S // QBLK):
                    qs = qb * QBLK
                    x_blk = x_ref[pl.ds(qs, QBLK), :]
                    q_raw = lax.dot_general(
                        x_blk, wq_p, (((1,), (0,)), ((), ())),
                        preferred_element_type=jnp.float32)
                    q_sw = lax.dot_general(
                        q_raw.astype(jnp.bfloat16), pv,
                        (((1,), (0,)), ((), ())),
                        preferred_element_type=jnp.float32)
                    q_p = ((q_raw * cos_f[qs:qs + QBLK, :]
                            + q_sw * sin_f[qs:qs + QBLK, :])
                           * SCALE).astype(jnp.bfloat16)

                    ctxs = []
                    for sub in range(2):
                        lo = sub * DH
                        q_h = q_p[:, lo:lo + DH]
                        k_h = k_p[:, lo:lo + DH]
                        v_h = v_p[:, lo:lo + DH]
                        s = lax.dot_general(
                            q_h, k_h, (((1,), (1,)), ((), ())),
                            preferred_element_type=jnp.float32)
                        e = jnp.exp(s)
                        den = jnp.sum(e, axis=-1, keepdims=True)
                        ctx = lax.dot_general(
                            e.astype(jnp.bfloat16), v_h,
                            (((1,), (0,)), ((), ())),
                            preferred_element_type=jnp.float32)
                        ctxs.append(ctx * (1.0 / den))
                    ctx_p = jnp.concatenate(ctxs, axis=1)
                    contrib = lax.dot_general(
                        ctx_p.astype(jnp.bfloat16), wo_p,
                        (((1,), (0,)), ((), ())),
                        preferred_element_type=jnp.float32)
                    out_ref[0, pl.ds(qs, QBLK), :] = (
                        out_ref[0, pl.ds(qs, QBLK), :] + contrib)
                return carry

            lax.fori_loop(0, H // 2, pair_body, 0)

            if h < N_DEV - 1:
                rdma.wait()
                if h == 1:
                    pl.semaphore_signal(credit_sem, inc=1, device_id=(left,),
                                        device_id_type=pl.DeviceIdType.MESH)

    return pl.pallas_call(
        body,
        out_shape=jax.ShapeDtypeStruct((1, S, D), jnp.float32),
        in_specs=[pl.BlockSpec(memory_space=pltpu.MemorySpace.VMEM)] * 5,
        out_specs=pl.BlockSpec(memory_space=pltpu.MemorySpace.VMEM),
        scratch_shapes=[
            pltpu.VMEM((2, 4, D, D), jnp.bfloat16),
            pltpu.SemaphoreType.DMA((N_DEV - 1,)),
            pltpu.SemaphoreType.DMA((N_DEV - 1,)),
            pltpu.SemaphoreType.REGULAR,
        ],
        compiler_params=pltpu.CompilerParams(
            collective_id=0,
            vmem_limit_bytes=100 * 1024 * 1024,
        ),
    )(x2, w_own, cos_t, sin_t, p_swap)
